# Initial kernel scaffold; baseline (speedup 1.0000x reference)
import jax
import jax.numpy as jnp
from jax.experimental import pallas as pl


def kernel(x, ids0, ids1, ids2, ids3):
    raise NotImplementedError("write your pallas kernel here")


# speedup vs baseline: 9.3263x; 9.3263x over previous
import functools

import jax
import jax.numpy as jnp
from jax import lax
from jax.experimental import pallas as pl
from jax.experimental.pallas import tpu as pltpu
from jax.experimental.pallas import tpu_sc as plsc

BATCH = 16384
DIM = 1024
GROUP = DIM // 4
ROW_BLOCK = 512
_LANES = 16


_CHUNK = 128


def _labels_sc_body(ids0_hbm, ids1_hbm, ids2_hbm, ids3_hbm, labels_hbm,
                    idx_v, val_v, sem):
    cid = lax.axis_index("c")
    sid = lax.axis_index("s")

    @pl.when(jnp.logical_and(cid == 0, sid == 0))
    def _():
        for g, ids_hbm in enumerate((ids0_hbm, ids1_hbm, ids2_hbm, ids3_hbm)):
            for i in range(GROUP // _LANES):
                val_v[pl.ds(i * _LANES, _LANES)] = jnp.full((_LANES,), g, jnp.int32)
            for c in range(GROUP // _CHUNK):
                pltpu.sync_copy(ids_hbm.at[pl.ds(c * _CHUNK, _CHUNK)], idx_v)
                pltpu.async_copy(
                    val_v.at[pl.ds(0, _CHUNK)], labels_hbm.at[idx_v], sem
                ).wait()


_labels_sc = pl.kernel(
    _labels_sc_body,
    out_type=jax.ShapeDtypeStruct((DIM,), jnp.int32),
    mesh=plsc.VectorSubcoreMesh(core_axis_name="c", subcore_axis_name="s"),
    scratch_types=[
        pltpu.VMEM((_CHUNK,), jnp.int32),
        pltpu.VMEM((GROUP,), jnp.int32),
        pltpu.SemaphoreType.DMA,
    ],
)


def _link_tc_body(lab_ref, x_ref, o_ref):
    x = x_ref[...]
    lab = lab_ref[...]
    is1 = lab == 1
    z = jnp.exp(jnp.where(is1, x, -jnp.abs(x)))
    recip = 1.0 / (1.0 + z)
    sig = jnp.where(x >= 0, recip, z * recip)
    sp = jnp.maximum(x, 0.0) + jnp.log1p(z)
    o_ref[...] = jnp.where(
        lab == 0, x, jnp.where(is1, z, jnp.where(lab == 2, sig, sp))
    )


_link_tc = pl.pallas_call(
    _link_tc_body,
    grid=(BATCH // ROW_BLOCK,),
    in_specs=[
        pl.BlockSpec((1, DIM), lambda i: (0, 0)),
        pl.BlockSpec((ROW_BLOCK, DIM), lambda i: (i, 0)),
    ],
    out_specs=pl.BlockSpec((ROW_BLOCK, DIM), lambda i: (i, 0)),
    out_shape=jax.ShapeDtypeStruct((BATCH, DIM), jnp.float32),
)


def kernel(x, ids0, ids1, ids2, ids3):
    labels = _labels_sc(
        ids0.astype(jnp.int32), ids1.astype(jnp.int32),
        ids2.astype(jnp.int32), ids3.astype(jnp.int32)
    )
    return _link_tc(labels.reshape(1, DIM), x)
